# Initial kernel scaffold; baseline (speedup 1.0000x reference)
import jax
import jax.numpy as jnp
from jax.experimental import pallas as pl


def kernel(queries, keys, values, adj, edge_pos_enc, Wq, bq, Wk, bk, Wv, bv, Wo, bo):
    raise NotImplementedError("write your pallas kernel here")


# speedup vs baseline: 9.0508x; 9.0508x over previous
import functools

import jax
import jax.numpy as jnp
from jax import lax
from jax.experimental import pallas as pl
from jax.experimental.pallas import tpu as pltpu
from jax.experimental.pallas import tpu_sc as plsc

L = 2048
D_IN = 1024
D_MDL = 1024
H = 16
DH = D_MDL // H
E = 65536
TEMP = 0.125


_ROWS = 256
_NT = L // _ROWS


def _proj_body(xq, xk, xv, wq, wk, wv, bq, bk, bv, oq, ok, ov):
    oq[...] = jnp.dot(xq[...], wq[...], preferred_element_type=jnp.float32, precision=lax.Precision.HIGHEST) + bq[...]
    ok[...] = jnp.dot(xk[...], wk[...], preferred_element_type=jnp.float32, precision=lax.Precision.HIGHEST) + bk[...]
    ov[...] = jnp.dot(xv[...], wv[...], preferred_element_type=jnp.float32, precision=lax.Precision.HIGHEST) + bv[...]


def _project(q2, k2, v2, Wq, Wk, Wv, bq, bk, bv):
    x_spec = pl.BlockSpec((_ROWS, D_IN), lambda i: (i, 0))
    w_spec = pl.BlockSpec((D_IN, D_MDL), lambda i: (0, 0))
    b_spec = pl.BlockSpec((1, D_MDL), lambda i: (0, 0))
    o_spec = pl.BlockSpec((_ROWS, D_MDL), lambda i: (i, 0))
    out_sd = jax.ShapeDtypeStruct((L, D_MDL), jnp.float32)
    return pl.pallas_call(
        _proj_body,
        grid=(_NT,),
        in_specs=[x_spec, x_spec, x_spec, w_spec, w_spec, w_spec,
                  b_spec, b_spec, b_spec],
        out_specs=[o_spec, o_spec, o_spec],
        out_shape=[out_sd, out_sd, out_sd],
    )(q2, k2, v2, Wq, Wk, Wv, bq, bk, bv)


_NS = 16
_EPW = E // _NS
_NCHUNK = _EPW // 128
_QROWS = 512
_QWORDS = _QROWS * L
_DUMP = _QWORDS
_ZW = 8192
_SLICE = _QWORDS // _NS


def _build_w_body(a0_hbm, a1_hbm, epe_hbm, out_hbm,
                  a0_v, a1_v, val_v, idx_v, zeros_v, flush_v, w_sh):
    c = lax.axis_index("c")
    s = lax.axis_index("s")
    base_e = s * _EPW

    pltpu.sync_copy(a0_hbm.at[pl.ds(base_e, _EPW)], a0_v)
    pltpu.sync_copy(a1_hbm.at[pl.ds(base_e, _EPW)], a1_v)
    pltpu.sync_copy(epe_hbm.at[pl.ds(base_e, _EPW)], val_v)

    def _val_step(t, carry):
        x = val_v[pl.ds(t * 16, 16)]
        val_v[pl.ds(t * 16, 16)] = jnp.exp(x * TEMP)
        return carry

    lax.fori_loop(0, _EPW // 16, _val_step, 0)

    def _zero_step(t, carry):
        zeros_v[pl.ds(t * 16, 16)] = jnp.zeros((16,), jnp.float32)
        return carry

    lax.fori_loop(0, _ZW // 16, _zero_step, 0)

    for p in range(2):
        base_row = c * (2 * _QROWS) + p * _QROWS

        for z in range(_SLICE // _ZW):
            pltpu.sync_copy(zeros_v,
                            w_sh.at[pl.ds(s * _SLICE + z * _ZW, _ZW)])
        plsc.subcore_barrier()

        for j in range(_NCHUNK):
            def _idx_step(t, carry, j=j):
                off = j * 128 + t * 16
                a0 = a0_v[pl.ds(off, 16)]
                a1 = a1_v[pl.ds(off, 16)]
                rel = a0 - base_row
                ok = (rel >= 0) & (rel < _QROWS)
                idx_v[j, pl.ds(t * 16, 16)] = jnp.where(
                    ok, rel * L + a1, _DUMP)
                return carry

            lax.fori_loop(0, 8, _idx_step, 0)
            pltpu.sync_copy(val_v.at[pl.ds(j * 128, 128)],
                            w_sh.at[idx_v.at[j]], add=True)

        def _dump_step(t, carry):
            idx_v[0, pl.ds(t * 16, 16)] = jnp.full((16,), _DUMP, jnp.int32)
            return carry

        lax.fori_loop(0, 8, _dump_step, 0)
        for _ in range(2):
            pltpu.sync_copy(zeros_v.at[pl.ds(0, 128)],
                            w_sh.at[idx_v.at[0]], add=True)
        pltpu.sync_copy(zeros_v.at[pl.ds(0, 16)],
                        w_sh.at[pl.ds(_QWORDS, 16)])
        pltpu.sync_copy(w_sh.at[pl.ds(_QWORDS, 16)], flush_v)
        plsc.subcore_barrier()
        out_base = base_row * L + s * _SLICE
        pltpu.sync_copy(w_sh.at[pl.ds(s * _SLICE, _SLICE)],
                        out_hbm.at[pl.ds(out_base, _SLICE)])
        plsc.subcore_barrier()


def _build_w(a0, a1, epe):
    mesh = plsc.VectorSubcoreMesh(core_axis_name="c", subcore_axis_name="s")
    return pl.kernel(
        _build_w_body,
        out_type=jax.ShapeDtypeStruct((L * L,), jnp.float32),
        mesh=mesh,
        scratch_types=[
            pltpu.VMEM((_EPW,), jnp.int32),
            pltpu.VMEM((_EPW,), jnp.int32),
            pltpu.VMEM((_EPW,), jnp.float32),
            pltpu.VMEM((_NCHUNK, 128), jnp.int32),
            pltpu.VMEM((_ZW,), jnp.float32),
            pltpu.VMEM((16,), jnp.float32),
            pltpu.VMEM_SHARED((_QWORDS + 16,), jnp.float32),
        ],
    )(a0, a1, epe)


def _attn_body(lq3, lk3, lv3, w_ref, wo3, bo_ref, out_ref):
    h = pl.program_id(1)
    q = lq3[0]
    k = lk3[0]
    v = lv3[0]
    s = lax.dot_general(q, k, (((1,), (1,)), ((), ())),
                        preferred_element_type=jnp.float32, precision=lax.Precision.HIGHEST) * TEMP
    m = jnp.max(s, axis=1, keepdims=True)
    p = w_ref[...] * jnp.exp(s - m)
    den = jnp.sum(p, axis=1, keepdims=True) + 1e-16
    ho = lax.dot_general(p, v, (((1,), (0,)), ((), ())),
                         preferred_element_type=jnp.float32, precision=lax.Precision.HIGHEST) / den
    contrib = jnp.dot(ho, wo3[0], preferred_element_type=jnp.float32, precision=lax.Precision.HIGHEST)

    @pl.when(h == 0)
    def _():
        out_ref[...] = contrib + bo_ref[...]

    @pl.when(h > 0)
    def _():
        out_ref[...] += contrib


def _attention(lq3, lk3, lv3, wmat, wo3, bo):
    return pl.pallas_call(
        _attn_body,
        grid=(_NT, H),
        in_specs=[
            pl.BlockSpec((1, _ROWS, DH), lambda i, h: (h, i, 0)),
            pl.BlockSpec((1, L, DH), lambda i, h: (h, 0, 0)),
            pl.BlockSpec((1, L, DH), lambda i, h: (h, 0, 0)),
            pl.BlockSpec((_ROWS, L), lambda i, h: (i, 0)),
            pl.BlockSpec((1, DH, D_MDL), lambda i, h: (h, 0, 0)),
            pl.BlockSpec((1, D_MDL), lambda i, h: (0, 0)),
        ],
        out_specs=pl.BlockSpec((_ROWS, D_MDL), lambda i, h: (i, 0)),
        out_shape=jax.ShapeDtypeStruct((L, D_MDL), jnp.float32),
    )(lq3, lk3, lv3, wmat, wo3, bo)


def kernel(queries, keys, values, adj, edge_pos_enc, Wq, bq, Wk, bk, Wv, bv,
           Wo, bo):
    q2 = queries.reshape(L, D_IN)
    k2 = keys.reshape(L, D_IN)
    v2 = values.reshape(L, D_IN)

    lq, lk, lv = _project(q2, k2, v2, Wq, Wk, Wv,
                          bq.reshape(1, D_MDL), bk.reshape(1, D_MDL),
                          bv.reshape(1, D_MDL))

    wflat = _build_w(adj[0], adj[1], edge_pos_enc)
    wmat = wflat.reshape(L, L)

    lq3 = lq.reshape(L, H, DH).transpose(1, 0, 2)
    lk3 = lk.reshape(L, H, DH).transpose(1, 0, 2)
    lv3 = lv.reshape(L, H, DH).transpose(1, 0, 2)
    wo3 = Wo.reshape(H, DH, D_MDL)

    out = _attention(lq3, lk3, lv3, wmat, wo3, bo.reshape(1, D_MDL))
    return out.reshape(1, L, D_MDL)


# speedup vs baseline: 26.4037x; 2.9173x over previous
import functools

import jax
import jax.numpy as jnp
from jax import lax
from jax.experimental import pallas as pl
from jax.experimental.pallas import tpu as pltpu
from jax.experimental.pallas import tpu_sc as plsc

L = 2048
D_IN = 1024
D_MDL = 1024
H = 16
DH = D_MDL // H
E = 65536
TEMP = 0.125


_ROWS = 256
_NT = L // _ROWS

def _split3(x):
    hi = x.astype(jnp.bfloat16)
    lo = (x - hi.astype(jnp.float32)).astype(jnp.bfloat16)
    return hi, lo


def _dot3(a, b, dims):
    ah, al = _split3(a)
    bh, bl = _split3(b)
    d = lambda x, y: lax.dot_general(x, y, dims,
                                     preferred_element_type=jnp.float32)
    return d(ah, bh) + d(ah, bl) + d(al, bh)


_MM_DIMS = (((1,), (0,)), ((), ()))
_QK_DIMS = (((1,), (1,)), ((), ()))


def _proj_body(xq, xk, xv, wq, wk, wv, bq, bk, bv,
               oqh, oql, okh, okl, ov):
    lq = (_dot3(xq[...], wq[...], _MM_DIMS) + bq[...]) * TEMP
    qh = lq.astype(jnp.bfloat16)
    oqh[...] = qh
    oql[...] = (lq - qh.astype(jnp.float32)).astype(jnp.bfloat16)
    lk = _dot3(xk[...], wk[...], _MM_DIMS) + bk[...]
    kh = lk.astype(jnp.bfloat16)
    okh[...] = kh
    okl[...] = (lk - kh.astype(jnp.float32)).astype(jnp.bfloat16)
    ov[...] = (_dot3(xv[...], wv[...], _MM_DIMS) + bv[...]).astype(jnp.bfloat16)


def _project(q2, k2, v2, Wq, Wk, Wv, bq, bk, bv):
    x_spec = pl.BlockSpec((_ROWS, D_IN), lambda i: (i, 0))
    w_spec = pl.BlockSpec((D_IN, D_MDL), lambda i: (0, 0))
    b_spec = pl.BlockSpec((1, D_MDL), lambda i: (0, 0))
    o_spec = pl.BlockSpec((_ROWS, D_MDL), lambda i: (i, 0))
    out_sd = jax.ShapeDtypeStruct((L, D_MDL), jnp.bfloat16)
    return pl.pallas_call(
        _proj_body,
        grid=(_NT,),
        in_specs=[x_spec, x_spec, x_spec, w_spec, w_spec, w_spec,
                  b_spec, b_spec, b_spec],
        out_specs=[o_spec] * 5,
        out_shape=[out_sd] * 5,
    )(q2, k2, v2, Wq, Wk, Wv, bq, bk, bv)


_NS = 16
_EPW = E // _NS
_NCHUNK = _EPW // 128
_QROWS = 512
_QWORDS = _QROWS * L
_DUMP = _QWORDS
_ZW = 8192
_SLICE = _QWORDS // _NS


def _build_w_body(a0_hbm, a1_hbm, epe_hbm, out_hbm,
                  a0_v, a1_v, val_v, idx_v, zeros_v, flush_v, w_sh):
    c = lax.axis_index("c")
    s = lax.axis_index("s")
    base_e = s * _EPW

    pltpu.sync_copy(a0_hbm.at[pl.ds(base_e, _EPW)], a0_v)
    pltpu.sync_copy(a1_hbm.at[pl.ds(base_e, _EPW)], a1_v)
    pltpu.sync_copy(epe_hbm.at[pl.ds(base_e, _EPW)], val_v)

    def _val_step(t, carry):
        x = val_v[pl.ds(t * 16, 16)]
        val_v[pl.ds(t * 16, 16)] = jnp.exp(x * TEMP)
        return carry

    lax.fori_loop(0, _EPW // 16, _val_step, 0)

    def _zero_step(t, carry):
        zeros_v[pl.ds(t * 16, 16)] = jnp.zeros((16,), jnp.float32)
        return carry

    lax.fori_loop(0, _ZW // 16, _zero_step, 0)

    for p in range(2):
        base_row = c * (2 * _QROWS) + p * _QROWS

        for z in range(_SLICE // _ZW):
            pltpu.sync_copy(zeros_v,
                            w_sh.at[pl.ds(s * _SLICE + z * _ZW, _ZW)])
        plsc.subcore_barrier()

        for j in range(_NCHUNK):
            def _idx_step(t, carry, j=j):
                off = j * 128 + t * 16
                a0 = a0_v[pl.ds(off, 16)]
                a1 = a1_v[pl.ds(off, 16)]
                rel = a0 - base_row
                ok = (rel >= 0) & (rel < _QROWS)
                idx_v[j, pl.ds(t * 16, 16)] = jnp.where(
                    ok, rel * L + a1, _DUMP)
                return carry

            lax.fori_loop(0, 8, _idx_step, 0)
            pltpu.sync_copy(val_v.at[pl.ds(j * 128, 128)],
                            w_sh.at[idx_v.at[j]], add=True)

        def _dump_step(t, carry):
            idx_v[0, pl.ds(t * 16, 16)] = jnp.full((16,), _DUMP, jnp.int32)
            return carry

        lax.fori_loop(0, 8, _dump_step, 0)
        for _ in range(2):
            pltpu.sync_copy(zeros_v.at[pl.ds(0, 128)],
                            w_sh.at[idx_v.at[0]], add=True)
        pltpu.sync_copy(zeros_v.at[pl.ds(0, 16)],
                        w_sh.at[pl.ds(_QWORDS, 16)])
        pltpu.sync_copy(w_sh.at[pl.ds(_QWORDS, 16)], flush_v)
        plsc.subcore_barrier()
        out_base = base_row * L + s * _SLICE
        pltpu.sync_copy(w_sh.at[pl.ds(s * _SLICE, _SLICE)],
                        out_hbm.at[pl.ds(out_base, _SLICE)])
        plsc.subcore_barrier()


def _build_w(a0, a1, epe):
    mesh = plsc.VectorSubcoreMesh(core_axis_name="c", subcore_axis_name="s")
    return pl.kernel(
        _build_w_body,
        out_type=jax.ShapeDtypeStruct((L * L,), jnp.float32),
        mesh=mesh,
        scratch_types=[
            pltpu.VMEM((_EPW,), jnp.int32),
            pltpu.VMEM((_EPW,), jnp.int32),
            pltpu.VMEM((_EPW,), jnp.float32),
            pltpu.VMEM((_NCHUNK, 128), jnp.int32),
            pltpu.VMEM((_ZW,), jnp.float32),
            pltpu.VMEM((16,), jnp.float32),
            pltpu.VMEM_SHARED((_QWORDS + 16,), jnp.float32),
        ],
    )(a0, a1, epe)


def _attn_body(qh, ql, kh, kl, w_ref, vaug, wo3, bo_ref, out_ref):
    h = pl.program_id(1)
    d = lambda a, b: lax.dot_general(a, b, _QK_DIMS,
                                     preferred_element_type=jnp.float32)
    s = d(qh[0], kh[h]) + d(qh[0], kl[h]) + d(ql[0], kh[h])
    p = (w_ref[...] * jnp.exp(s)).astype(jnp.bfloat16)
    pv = lax.dot_general(p, vaug[h], _MM_DIMS,
                         preferred_element_type=jnp.float32)
    den = pv[:, DH:DH + 1] + 1e-16
    ho = pv[:, :DH] / den
    contrib = lax.dot_general(ho.astype(jnp.bfloat16), wo3[0], _MM_DIMS,
                              preferred_element_type=jnp.float32)

    @pl.when(h == 0)
    def _():
        out_ref[...] = contrib + bo_ref[...]

    @pl.when(h > 0)
    def _():
        out_ref[...] += contrib


def _attention(qh3, ql3, kh3, kl3, wmat, vaug, wo3, bo):
    return pl.pallas_call(
        _attn_body,
        grid=(_NT, H),
        in_specs=[
            pl.BlockSpec((1, _ROWS, DH), lambda i, h: (h, i, 0)),
            pl.BlockSpec((1, _ROWS, DH), lambda i, h: (h, i, 0)),
            pl.BlockSpec((H, L, DH), lambda i, h: (0, 0, 0)),
            pl.BlockSpec((H, L, DH), lambda i, h: (0, 0, 0)),
            pl.BlockSpec((_ROWS, L), lambda i, h: (i, 0)),
            pl.BlockSpec((H, L, 128), lambda i, h: (0, 0, 0)),
            pl.BlockSpec((1, DH, D_MDL), lambda i, h: (h, 0, 0)),
            pl.BlockSpec((1, D_MDL), lambda i, h: (0, 0)),
        ],
        out_specs=pl.BlockSpec((_ROWS, D_MDL), lambda i, h: (i, 0)),
        out_shape=jax.ShapeDtypeStruct((L, D_MDL), jnp.float32),
    )(qh3, ql3, kh3, kl3, wmat, vaug, wo3, bo)


def kernel(queries, keys, values, adj, edge_pos_enc, Wq, bq, Wk, bk, Wv, bv,
           Wo, bo):
    q2 = queries.reshape(L, D_IN)
    k2 = keys.reshape(L, D_IN)
    v2 = values.reshape(L, D_IN)

    lqh, lql, lkh, lkl, lvb = _project(q2, k2, v2, Wq, Wk, Wv,
                                       bq.reshape(1, D_MDL),
                                       bk.reshape(1, D_MDL),
                                       bv.reshape(1, D_MDL))

    wflat = _build_w(adj[0], adj[1], edge_pos_enc)
    wmat = wflat.reshape(L, L)

    def _heads(x):
        return x.reshape(L, H, DH).transpose(1, 0, 2)

    vaug = jnp.concatenate(
        [_heads(lvb), jnp.ones((H, L, 1), jnp.bfloat16),
         jnp.zeros((H, L, 128 - DH - 1), jnp.bfloat16)], axis=-1)
    wo3 = Wo.astype(jnp.bfloat16).reshape(H, DH, D_MDL)

    out = _attention(_heads(lqh), _heads(lql), _heads(lkh), _heads(lkl),
                     wmat, vaug, wo3, bo.reshape(1, D_MDL))
    return out.reshape(1, L, D_MDL)


# speedup vs baseline: 26.9907x; 1.0222x over previous
import functools

import jax
import jax.numpy as jnp
from jax import lax
from jax.experimental import pallas as pl
from jax.experimental.pallas import tpu as pltpu
from jax.experimental.pallas import tpu_sc as plsc

L = 2048
D_IN = 1024
D_MDL = 1024
H = 16
DH = D_MDL // H
E = 65536
TEMP = 0.125


_ROWS = 256
_NT = L // _ROWS

def _split3(x):
    hi = x.astype(jnp.bfloat16)
    lo = (x - hi.astype(jnp.float32)).astype(jnp.bfloat16)
    return hi, lo


def _dot3(a, b, dims):
    ah, al = _split3(a)
    bh, bl = _split3(b)
    d = lambda x, y: lax.dot_general(x, y, dims,
                                     preferred_element_type=jnp.float32)
    return d(ah, bh) + d(ah, bl) + d(al, bh)


_MM_DIMS = (((1,), (0,)), ((), ()))
_QK_DIMS = (((1,), (1,)), ((), ()))


def _proj_body(xq, xk, xv, wq, wk, wv, bq, bk, bv,
               oqh, oql, okh, okl, ov):
    lq = (_dot3(xq[...], wq[...], _MM_DIMS) + bq[...]) * TEMP
    qh = lq.astype(jnp.bfloat16)
    oqh[...] = qh
    oql[...] = (lq - qh.astype(jnp.float32)).astype(jnp.bfloat16)
    lk = _dot3(xk[...], wk[...], _MM_DIMS) + bk[...]
    kh = lk.astype(jnp.bfloat16)
    okh[...] = kh
    okl[...] = (lk - kh.astype(jnp.float32)).astype(jnp.bfloat16)
    ov[...] = (_dot3(xv[...], wv[...], _MM_DIMS) + bv[...]).astype(jnp.bfloat16)


def _project(q2, k2, v2, Wq, Wk, Wv, bq, bk, bv):
    x_spec = pl.BlockSpec((_ROWS, D_IN), lambda i: (i, 0))
    w_spec = pl.BlockSpec((D_IN, D_MDL), lambda i: (0, 0))
    b_spec = pl.BlockSpec((1, D_MDL), lambda i: (0, 0))
    o_spec = pl.BlockSpec((_ROWS, D_MDL), lambda i: (i, 0))
    out_sd = jax.ShapeDtypeStruct((L, D_MDL), jnp.bfloat16)
    return pl.pallas_call(
        _proj_body,
        grid=(_NT,),
        in_specs=[x_spec, x_spec, x_spec, w_spec, w_spec, w_spec,
                  b_spec, b_spec, b_spec],
        out_specs=[o_spec] * 5,
        out_shape=[out_sd] * 5,
    )(q2, k2, v2, Wq, Wk, Wv, bq, bk, bv)


_NS = 16
_EPW = E // _NS
_NCHUNK = _EPW // 128
_QROWS = 512
_QWORDS = _QROWS * L
_DUMP = _QWORDS
_ZW = 8192
_SLICE = _QWORDS // _NS


def _build_w_body(a0_hbm, a1_hbm, epe_hbm, out_hbm,
                  a0_v, a1_v, val_v, idx_v, zeros_v, flush_v, w_sh, sem):
    c = lax.axis_index("c")
    s = lax.axis_index("s")
    base_e = s * _EPW

    cp0 = pltpu.async_copy(a0_hbm.at[pl.ds(base_e, _EPW)], a0_v, sem)
    cp1 = pltpu.async_copy(a1_hbm.at[pl.ds(base_e, _EPW)], a1_v, sem)
    cp2 = pltpu.async_copy(epe_hbm.at[pl.ds(base_e, _EPW)], val_v, sem)

    def _zero_step(t, carry):
        zeros_v[pl.ds(t * 16, 16)] = jnp.zeros((16,), jnp.float32)
        return carry

    lax.fori_loop(0, _ZW // 16, _zero_step, 0)
    cp0.wait()
    cp1.wait()
    cp2.wait()

    def _val_step(t, carry):
        x = val_v[pl.ds(t * 16, 16)]
        val_v[pl.ds(t * 16, 16)] = jnp.exp(x * TEMP)
        return carry

    lax.fori_loop(0, _EPW // 16, _val_step, 0)

    for p in range(2):
        base_row = c * (2 * _QROWS) + p * _QROWS

        zcps = [pltpu.async_copy(
                    zeros_v, w_sh.at[pl.ds(s * _SLICE + z * _ZW, _ZW)], sem)
                for z in range(_SLICE // _ZW)]
        def _idx_step(t, carry):
            a0 = a0_v[pl.ds(t * 16, 16)]
            a1 = a1_v[pl.ds(t * 16, 16)]
            rel = a0 - base_row
            ok = (rel >= 0) & (rel < _QROWS)
            idx_v[t // 8, pl.ds((t % 8) * 16, 16)] = jnp.where(
                ok, rel * L + a1, _DUMP)
            return carry

        lax.fori_loop(0, _EPW // 16, _idx_step, 0)
        def _dump_step(t, carry):
            idx_v[_NCHUNK, pl.ds(t * 16, 16)] = jnp.full(
                (16,), _DUMP, jnp.int32)
            return carry

        lax.fori_loop(0, 8, _dump_step, 0)
        for cp in zcps:
            cp.wait()
        plsc.subcore_barrier()

        scps = [pltpu.async_copy(val_v.at[pl.ds(j * 128, 128)],
                                 w_sh.at[idx_v.at[j]], sem, add=True)
                for j in range(_NCHUNK)]
        scps += [pltpu.async_copy(zeros_v.at[pl.ds(0, 128)],
                                  w_sh.at[idx_v.at[_NCHUNK]], sem, add=True)
                 for _ in range(2)]
        for cp in scps:
            cp.wait()
        pltpu.sync_copy(zeros_v.at[pl.ds(0, 16)],
                        w_sh.at[pl.ds(_QWORDS, 16)])
        pltpu.sync_copy(w_sh.at[pl.ds(_QWORDS, 16)], flush_v)
        plsc.subcore_barrier()
        out_base = base_row * L + s * _SLICE
        pltpu.sync_copy(w_sh.at[pl.ds(s * _SLICE, _SLICE)],
                        out_hbm.at[pl.ds(out_base, _SLICE)])
        plsc.subcore_barrier()


def _build_w(a0, a1, epe):
    mesh = plsc.VectorSubcoreMesh(core_axis_name="c", subcore_axis_name="s")
    return pl.kernel(
        _build_w_body,
        out_type=jax.ShapeDtypeStruct((L * L,), jnp.float32),
        mesh=mesh,
        scratch_types=[
            pltpu.VMEM((_EPW,), jnp.int32),
            pltpu.VMEM((_EPW,), jnp.int32),
            pltpu.VMEM((_EPW,), jnp.float32),
            pltpu.VMEM((_NCHUNK + 1, 128), jnp.int32),
            pltpu.VMEM((_ZW,), jnp.float32),
            pltpu.VMEM((16,), jnp.float32),
            pltpu.VMEM_SHARED((_QWORDS + 16,), jnp.float32),
            pltpu.SemaphoreType.DMA,
        ],
    )(a0, a1, epe)


def _attn_body(qh, ql, kh, kl, w_ref, vaug, wo3, bo_ref, out_ref):
    h = pl.program_id(1)
    d = lambda a, b: lax.dot_general(a, b, _QK_DIMS,
                                     preferred_element_type=jnp.float32)
    s = d(qh[0], kh[h]) + d(qh[0], kl[h]) + d(ql[0], kh[h])
    p = (w_ref[...] * jnp.exp(s)).astype(jnp.bfloat16)
    pv = lax.dot_general(p, vaug[h], _MM_DIMS,
                         preferred_element_type=jnp.float32)
    den = pv[:, DH:DH + 1] + 1e-16
    ho = pv[:, :DH] / den
    contrib = lax.dot_general(ho.astype(jnp.bfloat16), wo3[0], _MM_DIMS,
                              preferred_element_type=jnp.float32)

    @pl.when(h == 0)
    def _():
        out_ref[...] = contrib + bo_ref[...]

    @pl.when(h > 0)
    def _():
        out_ref[...] += contrib


def _attention(qh3, ql3, kh3, kl3, wmat, vaug, wo3, bo):
    return pl.pallas_call(
        _attn_body,
        grid=(_NT, H),
        in_specs=[
            pl.BlockSpec((1, _ROWS, DH), lambda i, h: (h, i, 0)),
            pl.BlockSpec((1, _ROWS, DH), lambda i, h: (h, i, 0)),
            pl.BlockSpec((H, L, DH), lambda i, h: (0, 0, 0)),
            pl.BlockSpec((H, L, DH), lambda i, h: (0, 0, 0)),
            pl.BlockSpec((_ROWS, L), lambda i, h: (i, 0)),
            pl.BlockSpec((H, L, 128), lambda i, h: (0, 0, 0)),
            pl.BlockSpec((1, DH, D_MDL), lambda i, h: (h, 0, 0)),
            pl.BlockSpec((1, D_MDL), lambda i, h: (0, 0)),
        ],
        out_specs=pl.BlockSpec((_ROWS, D_MDL), lambda i, h: (i, 0)),
        out_shape=jax.ShapeDtypeStruct((L, D_MDL), jnp.float32),
    )(qh3, ql3, kh3, kl3, wmat, vaug, wo3, bo)


def kernel(queries, keys, values, adj, edge_pos_enc, Wq, bq, Wk, bk, Wv, bv,
           Wo, bo):
    q2 = queries.reshape(L, D_IN)
    k2 = keys.reshape(L, D_IN)
    v2 = values.reshape(L, D_IN)

    lqh, lql, lkh, lkl, lvb = _project(q2, k2, v2, Wq, Wk, Wv,
                                       bq.reshape(1, D_MDL),
                                       bk.reshape(1, D_MDL),
                                       bv.reshape(1, D_MDL))

    wflat = _build_w(adj[0], adj[1], edge_pos_enc)
    wmat = wflat.reshape(L, L)

    def _heads(x):
        return x.reshape(L, H, DH).transpose(1, 0, 2)

    vaug = jnp.concatenate(
        [_heads(lvb), jnp.ones((H, L, 1), jnp.bfloat16),
         jnp.zeros((H, L, 128 - DH - 1), jnp.bfloat16)], axis=-1)
    wo3 = Wo.astype(jnp.bfloat16).reshape(H, DH, D_MDL)

    out = _attention(_heads(lqh), _heads(lql), _heads(lkh), _heads(lkl),
                     wmat, vaug, wo3, bo.reshape(1, D_MDL))
    return out.reshape(1, L, D_MDL)
